# Initial kernel scaffold; baseline (speedup 1.0000x reference)
import jax
import jax.numpy as jnp
from jax.experimental import pallas as pl


def kernel(x, edge_index, edge_attr, mol_batch, params):
    raise NotImplementedError("write your pallas kernel here")


# speedup vs baseline: 2.8692x; 2.8692x over previous
import functools

import jax
import jax.numpy as jnp
from jax import lax
from jax.experimental import pallas as pl
from jax.experimental.pallas import tpu as pltpu
from jax.experimental.pallas import tpu_sc as plsc

N = 10000
E = 320000
HID = 256
HALF = 128
NG = 64
NLAYERS = 3

NC = 2
NS = 16
EPT = E // NS
K = 80
NCHUNK = EPT // K
SROWS = 624
NREM = N - NS * SROWS

EB = 2000
NB = 2000


def _b_body(ea_ref, lew_ref, leb_ref, w1b_ref, mb1_ref, *out_refs):
    ea = ea_ref[...]
    d_ref = out_refs[-1]
    for l in range(NLAYERS):
        c = jnp.dot(lew_ref[l], w1b_ref[l], preferred_element_type=jnp.float32)
        d = (jnp.dot(leb_ref[l], w1b_ref[l], preferred_element_type=jnp.float32)
             + mb1_ref[l])
        b = jnp.dot(ea, c, preferred_element_type=jnp.float32) + d
        out_refs[l][0] = b[:, :HALF]
        out_refs[l][1] = b[:, HALF:]
        d_ref[l] = d


def _b_precompute(ea, lew, leb, w1b, mb1):
    nblk = E // EB
    full = lambda shape: pl.BlockSpec(shape, lambda i: (0,) * len(shape))
    outs = [jax.ShapeDtypeStruct((NC, E, HALF), jnp.float32)
            for _ in range(NLAYERS)]
    outs.append(jax.ShapeDtypeStruct((NLAYERS, 1, HID), jnp.float32))
    out_specs = [pl.BlockSpec((NC, EB, HALF), lambda i: (0, i, 0))
                 for _ in range(NLAYERS)]
    out_specs.append(full((NLAYERS, 1, HID)))
    return pl.pallas_call(
        _b_body,
        grid=(nblk,),
        in_specs=[
            pl.BlockSpec((EB, 16), lambda i: (i, 0)),
            full((NLAYERS, 16, HID)),
            full((NLAYERS, 1, HID)),
            full((NLAYERS, HID, HID)),
            full((NLAYERS, 1, HID)),
        ],
        out_specs=out_specs,
        out_shape=outs,
    )(ea, lew, leb, w1b, mb1)


def _s1_body(xin_ref, lw_ref, lb_ref, w1a_ref, xt_ref, a_ref):
    xt = (jnp.dot(xin_ref[...], lw_ref[...], preferred_element_type=jnp.float32)
          + lb_ref[...])
    xt_ref[...] = xt
    a_ref[...] = jnp.dot(xt, w1a_ref[...], preferred_element_type=jnp.float32)


def _stage1(xin, lw, lb, w1a):
    din = xin.shape[1]
    return pl.pallas_call(
        _s1_body,
        grid=(N // NB,),
        in_specs=[
            pl.BlockSpec((NB, din), lambda i: (i, 0)),
            pl.BlockSpec((din, HID), lambda i: (0, 0)),
            pl.BlockSpec((1, HID), lambda i: (0, 0)),
            pl.BlockSpec((HID, HID), lambda i: (0, 0)),
        ],
        out_specs=[
            pl.BlockSpec((NB, HID), lambda i: (i, 0)),
            pl.BlockSpec((NB, HID), lambda i: (i, 0)),
        ],
        out_shape=[
            jax.ShapeDtypeStruct((N, HID), jnp.float32),
            jax.ShapeDtypeStruct((N, HID), jnp.float32),
        ],
    )(xin, lw, lb, w1a)


def _edge_body(src_h, dst_h, a_h, b_h, z_h, s_out, idx_s, idx_d, h_v, g_v,
               S_sh, sem):
    c = lax.axis_index("c")
    s = lax.axis_index("s")
    row0 = s * SROWS

    pltpu.sync_copy(z_h, S_sh.at[pl.ds(row0, SROWS)])

    @pl.when(s == NS - 1)
    def _():
        pltpu.sync_copy(z_h.at[pl.ds(0, NREM)],
                        S_sh.at[pl.ds(NS * SROWS, NREM)])
    plsc.subcore_barrier()

    def chunk(j, carry):
        base = s * EPT + j * K
        pltpu.sync_copy(src_h.at[pl.ds(base, K)], idx_s)
        pltpu.sync_copy(dst_h.at[pl.ds(base, K)], idx_d)

        def fix_idx(q, carry2):
            sl = pl.ds(q * 16, 16)
            idx_s[sl] = idx_s[sl] * 2 + c
            return carry2
        lax.fori_loop(0, K // 16, fix_idx, 0)

        pltpu.sync_copy(b_h.at[c, pl.ds(base, K)], h_v)
        pltpu.async_copy(a_h.at[idx_s], g_v, sem).wait()

        def relu_row(r, carry2):
            for q in range(HALF // 16):
                sl = pl.ds(q * 16, 16)
                h_v[r, sl] = jnp.maximum(h_v[r, sl] + g_v[r, sl], 0.0)
            return carry2
        lax.fori_loop(0, K, relu_row, 0)

        pltpu.async_copy(h_v, S_sh.at[idx_d], sem, add=True).wait()
        return carry

    lax.fori_loop(0, NCHUNK, chunk, 0)
    plsc.subcore_barrier()

    pltpu.sync_copy(S_sh.at[pl.ds(row0, SROWS)],
                    s_out.at[c, pl.ds(row0, SROWS)])

    @pl.when(s == NS - 1)
    def _():
        pltpu.sync_copy(S_sh.at[pl.ds(NS * SROWS, NREM)],
                        s_out.at[c, pl.ds(NS * SROWS, NREM)])


_edge = pl.kernel(
    _edge_body,
    mesh=plsc.VectorSubcoreMesh(core_axis_name="c", subcore_axis_name="s"),
    out_type=[jax.ShapeDtypeStruct((NC, N, HALF), jnp.float32)],
    scratch_types=[
        pltpu.VMEM((K,), jnp.int32),
        pltpu.VMEM((K,), jnp.int32),
        pltpu.VMEM((K, HALF), jnp.float32),
        pltpu.VMEM((K, HALF), jnp.float32),
        pltpu.VMEM_SHARED((N, HALF), jnp.float32),
        pltpu.SemaphoreType.DMA,
    ],
)


def _cnt_body(dst_h, on_h, z_h, cnt_out, idx_d, ones_v, C_sh, sem):
    s = lax.axis_index("s")
    c = lax.axis_index("c")
    row0 = s * SROWS

    pltpu.sync_copy(z_h, C_sh.at[pl.ds(row0, SROWS)])

    @pl.when(s == NS - 1)
    def _():
        pltpu.sync_copy(z_h.at[pl.ds(0, NREM)],
                        C_sh.at[pl.ds(NS * SROWS, NREM)])
    pltpu.sync_copy(on_h, ones_v)
    plsc.subcore_barrier()

    def chunk(j, carry):
        base = s * EPT + j * K
        pltpu.sync_copy(dst_h.at[pl.ds(base, K)], idx_d)
        pltpu.async_copy(ones_v, C_sh.at[idx_d], sem, add=True).wait()
        return carry

    lax.fori_loop(0, NCHUNK, chunk, 0)
    plsc.subcore_barrier()

    pltpu.sync_copy(C_sh.at[pl.ds(row0, SROWS)],
                    cnt_out.at[c, pl.ds(row0, SROWS)])

    @pl.when(s == NS - 1)
    def _():
        pltpu.sync_copy(C_sh.at[pl.ds(NS * SROWS, NREM)],
                        cnt_out.at[c, pl.ds(NS * SROWS, NREM)])


_cnt = pl.kernel(
    _cnt_body,
    mesh=plsc.VectorSubcoreMesh(core_axis_name="c", subcore_axis_name="s"),
    out_type=[jax.ShapeDtypeStruct((NC, N, HALF), jnp.float32)],
    scratch_types=[
        pltpu.VMEM((K,), jnp.int32),
        pltpu.VMEM((K, HALF), jnp.float32),
        pltpu.VMEM_SHARED((N, HALF), jnp.float32),
        pltpu.SemaphoreType.DMA,
    ],
)


def _make_s2_body(has_skip):
    def body(s_ref, a_ref, xt_ref, cnt_ref, leb_ref, w1b_ref,
             mb1_ref, mw2a_ref, mw2b_ref, mb2_ref, uw1a_ref, uw1b_ref,
             ub1_ref, uw2_ref, ub2_ref, *rest):
        if has_skip:
            prev_ref, sk_ref, out_ref = rest
        else:
            out_ref = rest[0]
        d = (jnp.dot(leb_ref[...], w1b_ref[...],
                     preferred_element_type=jnp.float32) + mb1_ref[...])
        a = a_ref[...]
        sc0 = s_ref[0] + jnp.maximum(a[:, :HALF] + d[:, :HALF], 0.0)
        sc1 = s_ref[1] + jnp.maximum(a[:, HALF:] + d[:, HALF:], 0.0)
        aggr = (jnp.dot(sc0, mw2a_ref[...], preferred_element_type=jnp.float32)
                + jnp.dot(sc1, mw2b_ref[...], preferred_element_type=jnp.float32)
                + (cnt_ref[...][:, 0:1] + 1.0) * mb2_ref[...])
        u1 = jnp.maximum(
            jnp.dot(aggr, uw1a_ref[...], preferred_element_type=jnp.float32)
            + jnp.dot(xt_ref[...], uw1b_ref[...],
                      preferred_element_type=jnp.float32)
            + ub1_ref[...], 0.0)
        out = jnp.dot(u1, uw2_ref[...], preferred_element_type=jnp.float32) \
            + ub2_ref[...]
        if has_skip:
            out = out + jnp.maximum(sk_ref[...][0:1, 0:1], 0.0) * prev_ref[...]
        out_ref[...] = out
    return body


def _stage2(s_all, a, xt, cnt, leb, w1b, mb1, mw2a, mw2b, mb2,
            uw1a, uw1b, ub1, uw2, ub2, prev=None, sk=None):
    has_skip = prev is not None
    full = lambda shape: pl.BlockSpec(shape, lambda i: (0,) * len(shape))
    in_specs = [
        pl.BlockSpec((NC, NB, HALF), lambda i: (0, i, 0)),
        pl.BlockSpec((NB, HID), lambda i: (i, 0)),
        pl.BlockSpec((NB, HID), lambda i: (i, 0)),
        pl.BlockSpec((NB, HALF), lambda i: (i, 0)),
        full((1, HID)),
        full((HID, HID)),
        full((1, HID)),
        full((HALF, HID)),
        full((HALF, HID)),
        full((1, HID)),
        full((HID, HID)),
        full((HID, HID)),
        full((1, HID)),
        full((HID, HID)),
        full((1, HID)),
    ]
    args = [s_all, a, xt, cnt, leb, w1b, mb1, mw2a, mw2b, mb2,
            uw1a, uw1b, ub1, uw2, ub2]
    if has_skip:
        in_specs += [pl.BlockSpec((NB, HID), lambda i: (i, 0)),
                     full((1, HALF))]
        args += [prev, sk]
    return pl.pallas_call(
        _make_s2_body(has_skip),
        grid=(N // NB,),
        in_specs=in_specs,
        out_specs=pl.BlockSpec((NB, HID), lambda i: (i, 0)),
        out_shape=jax.ShapeDtypeStruct((N, HID), jnp.float32),
    )(*args)


def _pool_body(emb_ref, mb_ref, out_ref):
    mb = mb_ref[...]
    gid = lax.broadcasted_iota(jnp.int32, (N, NG), 1)
    oh = (mb == gid).astype(jnp.float32)
    sums = lax.dot_general(oh, emb_ref[...], (((0,), (0,)), ((), ())),
                           preferred_element_type=jnp.float32)
    counts = jnp.sum(oh, axis=0)[:, None]
    out_ref[...] = sums / jnp.maximum(counts, 1.0)


def _pool(emb, mb):
    return pl.pallas_call(
        _pool_body,
        in_specs=[pl.BlockSpec((N, HID), lambda: (0, 0)),
                  pl.BlockSpec((N, 1), lambda: (0, 0))],
        out_specs=pl.BlockSpec((NG, HID), lambda: (0, 0)),
        out_shape=jax.ShapeDtypeStruct((NG, HID), jnp.float32),
    )(emb, mb)


def kernel(x, edge_index, edge_attr, mol_batch, params):
    layers = params["layers"]
    src = edge_index[0].astype(jnp.int32)
    dst = edge_index[1].astype(jnp.int32)

    lew = jnp.stack([p["le_W"] for p in layers])
    leb = jnp.stack([p["le_b"][None, :] for p in layers])
    w1b = jnp.stack([p["m_W1"][HID:] for p in layers])
    mb1 = jnp.stack([p["m_b1"][None, :] for p in layers])

    b0, b1, b2, _dvec = _b_precompute(edge_attr, lew, leb, w1b, mb1)
    bstack = [b0, b1, b2]

    zeros = jnp.zeros((SROWS, HALF), jnp.float32)
    ones = jnp.ones((K, HALF), jnp.float32)

    (cnt_all,) = _cnt(dst, ones, zeros)
    cnt = cnt_all[0]

    xin = x
    for l in range(NLAYERS):
        p = layers[l]
        xt, a = _stage1(xin, p["ln_W"], p["ln_b"][None, :], p["m_W1"][:HID])
        a_flat = a.reshape(2 * N, HALF)
        (s_all,) = _edge(src, dst, a_flat, bstack[l], zeros)
        s2_args = (s_all, a, xt, cnt,
                   p["le_b"][None, :], p["m_W1"][HID:], p["m_b1"][None, :],
                   p["m_W2"][:HALF], p["m_W2"][HALF:], p["m_b2"][None, :],
                   p["u_W1"][:HID], p["u_W1"][HID:], p["u_b1"][None, :],
                   p["u_W2"], p["u_b2"][None, :])
        if l == 0:
            xin = _stage2(*s2_args)
        else:
            sk = jnp.full((1, HALF), params["skip"][l - 1], jnp.float32)
            xin = _stage2(*s2_args, prev=xin, sk=sk)

    comp = _pool(xin, mol_batch.astype(jnp.int32)[:, None])
    return xin, comp
